# baseline (speedup 1.0000x reference)
import jax
import jax.numpy as jnp
from jax import lax
from jax.experimental import pallas as pl
from jax.experimental.pallas import tpu as pltpu

_NBUF = 8
_DCHUNK = 64


def _body(row_ref, col_ref, x_ref, o_ref, in_bufs, out_bufs, pos_ref,
          in_sems, out_sems):
    n_chunks = x_ref.shape[0]
    n_dc = pos_ref.shape[0] // _DCHUNK

    def in_copy(t, slot):
        return pltpu.make_async_copy(x_ref.at[t], in_bufs.at[slot],
                                     in_sems.at[slot])

    def out_copy(t, slot):
        return pltpu.make_async_copy(out_bufs.at[slot], o_ref.at[t],
                                     out_sems.at[slot])

    for i in range(_NBUF):
        in_copy(i, i).start()

    row_t = row_ref[...].T
    col_t = col_ref[...].T
    pos3 = row_t[:, :, None] + col_t[:, None, :]
    pos_ref[...] = pos3.reshape(pos_ref.shape)

    def step(t, carry):
        slot = lax.rem(t, _NBUF)
        dc = lax.rem(t, n_dc)
        in_copy(t, slot).wait()

        @pl.when(t >= _NBUF)
        def _():
            out_copy(t, slot).wait()

        out_bufs[slot] = in_bufs[slot] + pos_ref[pl.ds(dc * _DCHUNK, _DCHUNK), :]
        out_copy(t, slot).start()

        @pl.when(t + _NBUF < n_chunks)
        def _():
            in_copy(t + _NBUF, slot).start()

        return carry

    lax.fori_loop(0, n_chunks, step, 0)
    for i in range(_NBUF):
        out_copy(n_chunks - _NBUF + i, i).wait()


def kernel(x, row_embed, col_embed):
    B, D, H, W = x.shape
    n_dc = D // _DCHUNK
    xf = x.reshape(B * n_dc, _DCHUNK, H * W)
    out = pl.pallas_call(
        _body,
        in_specs=[
            pl.BlockSpec(memory_space=pltpu.MemorySpace.VMEM),
            pl.BlockSpec(memory_space=pltpu.MemorySpace.VMEM),
            pl.BlockSpec(memory_space=pltpu.MemorySpace.HBM),
        ],
        out_specs=pl.BlockSpec(memory_space=pltpu.MemorySpace.HBM),
        out_shape=jax.ShapeDtypeStruct(xf.shape, x.dtype),
        scratch_shapes=[
            pltpu.VMEM((_NBUF, _DCHUNK, H * W), jnp.float32),
            pltpu.VMEM((_NBUF, _DCHUNK, H * W), jnp.float32),
            pltpu.VMEM((D, H * W), jnp.float32),
            pltpu.SemaphoreType.DMA((_NBUF,)),
            pltpu.SemaphoreType.DMA((_NBUF,)),
        ],
    )(row_embed, col_embed, xf)
    return out.reshape(B, D, H, W)


# speedup vs baseline: 1.1929x; 1.1929x over previous
import functools

import jax
import jax.numpy as jnp
from jax import lax
from jax.experimental import pallas as pl
from jax.experimental.pallas import tpu as pltpu
from jax.experimental.pallas import tpu_sc as plsc

_NC, _NS = 2, 16
_NW = _NC * _NS
_LANES = 16


def _pos_body(row_ref, col_ref, pos_ref):
    row_t = row_ref[...].T
    col_t = col_ref[...].T
    pos3 = row_t[:, :, None] + col_t[:, None, :]
    pos_ref[...] = pos3.reshape(pos_ref.shape)


def _build_pos(row_embed, col_embed):
    H, D = row_embed.shape
    W = col_embed.shape[0]
    return pl.pallas_call(
        _pos_body,
        out_shape=jax.ShapeDtypeStruct((D, H * W), jnp.float32),
    )(row_embed, col_embed)


def _sc_add_kernel(B, D, HW):
    d_per_w = D // _NW
    rows = d_per_w // 2
    n_chunks = 2 * B

    mesh = plsc.VectorSubcoreMesh(core_axis_name="c", subcore_axis_name="s")

    @functools.partial(
        pl.kernel,
        out_type=jax.ShapeDtypeStruct((B, D, HW), jnp.float32),
        mesh=mesh,
        scratch_types=[
            pltpu.VMEM((d_per_w, HW), jnp.float32),
            pltpu.VMEM((rows, HW), jnp.float32),
            pltpu.VMEM((rows, HW), jnp.float32),
            pltpu.VMEM((rows, HW), jnp.float32),
            pltpu.VMEM((rows, HW), jnp.float32),
            pltpu.SemaphoreType.DMA,
            pltpu.SemaphoreType.DMA,
            pltpu.SemaphoreType.DMA,
            pltpu.SemaphoreType.DMA,
        ],
    )
    def k(x_hbm, pos_hbm, out_hbm, pos_v, in0, in1, out0, out1,
          si0, si1, so0, so1):
        w = lax.axis_index("s") * _NC + lax.axis_index("c")
        d0 = w * d_per_w
        pltpu.sync_copy(pos_hbm.at[pl.ds(d0, d_per_w)], pos_v)

        ins = (in0, in1)
        outs = (out0, out1)
        sis = (si0, si1)
        sos = (so0, so1)

        def in_copy(t, s):
            b, half = t // 2, t % 2
            return pltpu.make_async_copy(
                x_hbm.at[b, pl.ds(d0 + half * rows, rows)], ins[s], sis[s])

        def out_copy(t, s):
            b, half = t // 2, t % 2
            return pltpu.make_async_copy(
                outs[s], out_hbm.at[b, pl.ds(d0 + half * rows, rows)], sos[s])

        in_copy(0, 0).start()
        in_copy(1, 1).start()
        for t in range(n_chunks):
            s = t % 2
            half = t % 2
            in_copy(t, s).wait()
            if t >= 2:
                out_copy(t - 2, s).wait()

            for r in range(rows):
                pr = half * rows + r

                def body(c, carry, r=r, pr=pr, s=s):
                    sl = pl.ds(c * _LANES, _LANES)
                    outs[s][r, sl] = ins[s][r, sl] + pos_v[pr, sl]
                    return carry

                lax.fori_loop(0, HW // _LANES, body, 0)

            out_copy(t, s).start()
            if t + 2 < n_chunks:
                in_copy(t + 2, s).start()
        out_copy(n_chunks - 2, 0).wait()
        out_copy(n_chunks - 1, 1).wait()

    return k


def kernel(x, row_embed, col_embed):
    B, D, H, W = x.shape
    HW = H * W
    pos = _build_pos(row_embed, col_embed)
    xf = x.reshape(B, D, HW)
    out = _sc_add_kernel(B, D, HW)(xf, pos)
    return out.reshape(B, D, H, W)


# speedup vs baseline: 1.8812x; 1.5770x over previous
import functools

import jax
import jax.numpy as jnp
from jax import lax
from jax.experimental import pallas as pl
from jax.experimental.pallas import tpu as pltpu
from jax.experimental.pallas import tpu_sc as plsc

_NC, _NS = 2, 16
_NW = _NC * _NS
_LANES = 16
_NBUF = 4


def _pos_body(row_ref, col_ref, pos_ref):
    row_t = row_ref[...].T
    col_t = col_ref[...].T
    pos3 = row_t[:, :, None] + col_t[:, None, :]
    pos_ref[...] = pos3.reshape(pos_ref.shape)


def _build_pos(row_embed, col_embed):
    H, D = row_embed.shape
    W = col_embed.shape[0]
    return pl.pallas_call(
        _pos_body,
        out_shape=jax.ShapeDtypeStruct((D, H * W), jnp.float32),
    )(row_embed, col_embed)


def _sc_add_kernel(B, D, HW):
    d_per_w = D // _NW
    rows = d_per_w // 2
    n_chunks = 2 * B

    mesh = plsc.VectorSubcoreMesh(core_axis_name="c", subcore_axis_name="s")

    @functools.partial(
        pl.kernel,
        out_type=jax.ShapeDtypeStruct((B, D, HW), jnp.float32),
        mesh=mesh,
        scratch_types=[
            pltpu.VMEM((d_per_w, HW), jnp.float32),
            [pltpu.VMEM((rows, HW), jnp.float32) for _ in range(_NBUF)],
            [pltpu.SemaphoreType.DMA for _ in range(_NBUF)],
            [pltpu.SemaphoreType.DMA for _ in range(_NBUF)],
        ],
    )
    def k(x_hbm, pos_hbm, out_hbm, pos_v, bufs, sis, sos):
        w = lax.axis_index("s") * _NC + lax.axis_index("c")
        d0 = w * d_per_w

        def in_copy(t):
            b, half, s = t // 2, t % 2, t % _NBUF
            return pltpu.make_async_copy(
                x_hbm.at[b, pl.ds(d0 + half * rows, rows)], bufs[s], sis[s])

        def out_copy(t):
            b, half, s = t // 2, t % 2, t % _NBUF
            return pltpu.make_async_copy(
                bufs[s], out_hbm.at[b, pl.ds(d0 + half * rows, rows)], sos[s])

        in_copy(0).start()
        in_copy(1).start()
        pltpu.sync_copy(pos_hbm.at[pl.ds(d0, d_per_w)], pos_v)

        for t in range(n_chunks):
            s = t % _NBUF
            half = t % 2
            in_copy(t).wait()

            for r in range(rows):
                pr = half * rows + r

                def _body(i, r=r, pr=pr, s=s):
                    sl = pl.ds(i, _LANES)
                    plsc.addupdate(bufs[s].at[r, sl], pos_v[pr, sl])

                plsc.parallel_loop(0, HW, step=16, unroll=8)(_body)

            out_copy(t).start()
            if t + 2 < n_chunks:
                if t >= 2:
                    out_copy(t - 2).wait()
                in_copy(t + 2).start()
        for t in range(n_chunks - _NBUF, n_chunks):
            out_copy(t).wait()

    return k


def kernel(x, row_embed, col_embed):
    B, D, H, W = x.shape
    HW = H * W
    pos = _build_pos(row_embed, col_embed)
    xf = x.reshape(B, D, HW)
    out = _sc_add_kernel(B, D, HW)(xf, pos)
    return out.reshape(B, D, H, W)


# speedup vs baseline: 1.9292x; 1.0255x over previous
import functools

import jax
import jax.numpy as jnp
from jax import lax
from jax.experimental import pallas as pl
from jax.experimental.pallas import tpu as pltpu
from jax.experimental.pallas import tpu_sc as plsc

_NC, _NS = 2, 16
_NW = _NC * _NS
_LANES = 16
_NBUF = 4


def _pos_body(row_ref, col_ref, pos_ref):
    row_t = row_ref[...].T
    col_t = col_ref[...].T
    pos3 = row_t[:, :, None] + col_t[:, None, :]
    pos_ref[...] = pos3.reshape(pos_ref.shape)


def _build_pos(row_embed, col_embed):
    H, D = row_embed.shape
    W = col_embed.shape[0]
    return pl.pallas_call(
        _pos_body,
        out_shape=jax.ShapeDtypeStruct((D, H * W), jnp.float32),
    )(row_embed, col_embed)


def _sc_add_kernel(B, D, HW):
    d_per_w = D // _NW
    rows = d_per_w // 2
    n_chunks = 2 * B

    mesh = plsc.VectorSubcoreMesh(core_axis_name="c", subcore_axis_name="s")

    @functools.partial(
        pl.kernel,
        out_type=jax.ShapeDtypeStruct((B, D, HW), jnp.float32),
        mesh=mesh,
        scratch_types=[
            pltpu.VMEM((d_per_w, HW), jnp.float32),
            pltpu.VMEM((_NBUF, rows, HW), jnp.float32),
            pltpu.SemaphoreType.DMA((_NBUF,)),
            pltpu.SemaphoreType.DMA((_NBUF,)),
        ],
    )
    def k(x_hbm, pos_hbm, out_hbm, pos_v, bufs, sis, sos):
        w = lax.axis_index("s") * _NC + lax.axis_index("c")
        d0 = w * d_per_w

        def in_copy(t):
            b, half, s = t // 2, t % 2, t % _NBUF
            return pltpu.make_async_copy(
                x_hbm.at[b, pl.ds(d0 + half * rows, rows)],
                bufs.at[s], sis.at[s])

        def out_copy(t):
            b, half, s = t // 2, t % 2, t % _NBUF
            return pltpu.make_async_copy(
                bufs.at[s],
                out_hbm.at[b, pl.ds(d0 + half * rows, rows)], sos.at[s])

        in_copy(0).start()
        in_copy(1).start()
        pltpu.sync_copy(pos_hbm.at[pl.ds(d0, d_per_w)], pos_v)

        def step(t, carry):
            s = t % _NBUF
            half = t % 2
            in_copy(t).wait()

            for r in range(rows):
                def _body(i, r=r, s=s, half=half):
                    sl = pl.ds(i, _LANES)
                    plsc.addupdate(bufs.at[s, r, sl],
                                   pos_v[half * rows + r, sl])

                plsc.parallel_loop(0, HW, step=16, unroll=8)(_body)

            out_copy(t).start()

            @pl.when(t + 2 < n_chunks)
            def _():
                @pl.when(t >= 2)
                def _():
                    out_copy(t - 2).wait()
                in_copy(t + 2).start()

            return carry

        lax.fori_loop(0, n_chunks, step, 0)
        for t in range(n_chunks - _NBUF, n_chunks):
            out_copy(t).wait()

    return k


def kernel(x, row_embed, col_embed):
    B, D, H, W = x.shape
    HW = H * W
    pos = _build_pos(row_embed, col_embed)
    xf = x.reshape(B, D, HW)
    out = _sc_add_kernel(B, D, HW)(xf, pos)
    return out.reshape(B, D, H, W)
